# baseline (speedup 1.0000x reference)
import functools
import jax
import jax.numpy as jnp
from jax import lax
from jax.experimental import pallas as pl
from jax.experimental.pallas import tpu as pltpu
from jax.experimental.pallas import tpu_sc as plsc

N_NODE = 50000
N_PAD = 50176
EMB = 64
HALF = 32
NNZ = 800000
BN = 1024
NSUB = 16
E_PER_S = NNZ // NSUB
CH = 400
N_CHUNK = E_PER_S // CH
ROWS_PER_S = N_PAD // NSUB
ZB = 392


def _dense_body(side_ref, ego_ref, wg_ref, bg_ref, wb_ref, bb_ref,
                ego_out_ref, norm_out_ref):
    side = side_ref[...]
    ego = ego_ref[...]
    sum_emb = jnp.dot(side, wg_ref[...],
                      preferred_element_type=jnp.float32) + bg_ref[...]
    bi = jnp.dot(ego * side, wb_ref[...],
                 preferred_element_type=jnp.float32) + bb_ref[...]
    x = sum_emb + bi
    x = jnp.where(x > 0, x, 0.2 * x)
    ego_out_ref[...] = x
    nrm = jnp.maximum(
        jnp.sqrt(jnp.sum(x * x, axis=1, keepdims=True)), 1e-12)
    norm_out_ref[...] = x / nrm


def _dense_layer(side, ego, wg, bg, wb, bb):
    n = side.shape[0]
    grid = (n // BN,)
    row_spec = pl.BlockSpec((BN, EMB), lambda i: (i, 0))
    w_spec = pl.BlockSpec((EMB, EMB), lambda i: (0, 0))
    b_spec = pl.BlockSpec((1, EMB), lambda i: (0, 0))
    return pl.pallas_call(
        _dense_body,
        grid=grid,
        in_specs=[row_spec, row_spec, w_spec, b_spec, w_spec, b_spec],
        out_specs=[row_spec, row_spec],
        out_shape=[
            jax.ShapeDtypeStruct((n, EMB), jnp.float32),
            jax.ShapeDtypeStruct((n, EMB), jnp.float32),
        ],
    )(side, ego, wg, bg, wb, bb)


NSLOT = 6


def _spmm_body(ego4_hbm, colq_hbm, row_hbm, val_hbm, out_hbm,
               col_v, dest_v, val_v, rows_v, zz_v, acc_sh,
               sem_t0, sem_t1, sem_t2, sem_t3, sem_t4, sem_t5,
               sem_g0, sem_g1, sem_g2, sem_g3, sem_g4, sem_g5,
               sem_s0, sem_s1, sem_s2, sem_s3, sem_s4, sem_s5):
    c = lax.axis_index("c")
    s = lax.axis_index("s")
    row0 = s * ROWS_PER_S
    ebase = s * E_PER_S
    sem_t = (sem_t0, sem_t1, sem_t2, sem_t3, sem_t4, sem_t5)
    sem_g = (sem_g0, sem_g1, sem_g2, sem_g3, sem_g4, sem_g5)
    sem_s = (sem_s0, sem_s1, sem_s2, sem_s3, sem_s4, sem_s5)

    for p in range(2):
        q = c * 2 + p

        def zero_body(i, _):
            zz_v[i, 0:16] = jnp.zeros((16,), jnp.float32)
            return 0
        lax.fori_loop(0, ZB, zero_body, 0)

        def zcopy_body(t, _):
            pltpu.sync_copy(zz_v, acc_sh.at[pl.ds(row0 + t * ZB, ZB)])
            return 0
        lax.fori_loop(0, ROWS_PER_S // ZB, zcopy_body, 0)
        plsc.subcore_barrier()

        def wait_scatter(sl):
            pltpu.make_async_copy(row_hbm.at[pl.ds(0, CH)],
                                  rows_v.at[sl], sem_s[sl]).wait()

        def stage(i, sl):
            off = ebase + i * CH
            pltpu.async_copy(colq_hbm.at[q, pl.ds(off, CH)], col_v.at[sl],
                             sem_t[sl])
            pltpu.async_copy(row_hbm.at[pl.ds(off, CH)], dest_v.at[sl],
                             sem_t[sl])
            pltpu.async_copy(val_hbm.at[pl.ds(off, CH)], val_v.at[sl],
                             sem_t[sl])

        def gather_start(sl):
            for dst in (col_v, dest_v, val_v):
                pltpu.make_async_copy(row_hbm.at[pl.ds(0, CH)],
                                      dst.at[sl], sem_t[sl]).wait()
            pltpu.async_copy(ego4_hbm.at[col_v.at[sl]], rows_v.at[sl],
                             sem_g[sl])

        def consume(sl):
            pltpu.make_async_copy(row_hbm.at[pl.ds(0, CH)],
                                  rows_v.at[sl], sem_g[sl]).wait()

            @plsc.parallel_loop(0, CH // 16, 1, unroll=4)
            def _(g):
                e0 = g * 16
                vals16 = val_v[sl, pl.ds(e0, 16)]
                for j in range(16):
                    rows_v[sl, e0 + j, 0:16] = (
                        rows_v[sl, e0 + j, 0:16] * vals16[j])

            pltpu.async_copy(rows_v.at[sl], acc_sh.at[dest_v.at[sl]],
                             sem_s[sl], add=True)

        stage(0, 0)
        stage(1, 1)
        gather_start(0)

        def ring_body(t, _):
            i0 = t * NSLOT
            for sl in range(NSLOT):
                i = i0 + sl
                nsl = (sl + 1) % NSLOT
                psl = (sl + 2) % NSLOT

                @pl.when(i + 2 < N_CHUNK)
                def _():
                    @pl.when(i >= NSLOT - 2)
                    def _():
                        wait_scatter(psl)
                    stage(i + 2, psl)

                @pl.when(i + 1 < N_CHUNK)
                def _():
                    gather_start(nsl)

                @pl.when(i < N_CHUNK)
                def _():
                    consume(sl)
            return 0

        lax.fori_loop(0, (N_CHUNK + NSLOT - 1) // NSLOT, ring_body, 0)
        for sl in range(NSLOT):
            wait_scatter(sl)
        plsc.subcore_barrier()

        pltpu.sync_copy(acc_sh.at[pl.ds(row0, ROWS_PER_S)],
                        out_hbm.at[pl.ds(row0, ROWS_PER_S), q])
        plsc.subcore_barrier()


_spmm = functools.partial(
    pl.kernel,
    out_type=jax.ShapeDtypeStruct((N_PAD, 4, 16), jnp.float32),
    mesh=plsc.VectorSubcoreMesh(core_axis_name="c", subcore_axis_name="s"),
    scratch_types=[
        pltpu.VMEM((NSLOT, CH), jnp.int32),
        pltpu.VMEM((NSLOT, CH), jnp.int32),
        pltpu.VMEM((NSLOT, CH), jnp.float32),
        pltpu.VMEM((NSLOT, CH, 16), jnp.float32),
        pltpu.VMEM((ZB, 16), jnp.float32),
        pltpu.VMEM_SHARED((N_PAD, 16), jnp.float32),
    ] + [pltpu.SemaphoreType.DMA] * 18,
    compiler_params=pltpu.CompilerParams(use_tc_tiling_on_sc=False),
)(_spmm_body)


def _spmm_layer(ego, colq_all, adj_row, adj_val):
    ego4 = ego.reshape(4 * N_PAD, 16)
    out = _spmm(ego4, colq_all, adj_row, adj_val)
    return out.reshape(N_PAD, EMB)


B_OUT = 4096
B_PER_W = B_OUT // 32


def _fgather_body(uemb_hbm, iemb_hbm, n1_hbm, n2_hbm, n3_hbm,
                  users_hbm, pos_hbm, neg_hbm,
                  ug_hbm, pg_hbm, ng_hbm,
                  idx_v, idxo_v, buf_v, sem):
    c = lax.axis_index("c")
    s = lax.axis_index("s")
    wid = s * 2 + c
    base = wid * B_PER_W

    for idx_hbm, out_hbm, is_item in ((users_hbm, ug_hbm, False),
                                      (pos_hbm, pg_hbm, True),
                                      (neg_hbm, ng_hbm, True)):
        pltpu.sync_copy(idx_hbm.at[pl.ds(base, B_PER_W)], idx_v)
        emb = iemb_hbm if is_item else uemb_hbm
        pltpu.async_copy(emb.at[idx_v], buf_v, sem).wait()
        pltpu.sync_copy(buf_v, out_hbm.at[pl.ds(base, B_PER_W), pl.ds(0, EMB)])
        off = 25000 if is_item else 0

        def off_body(g, _):
            idxo_v[pl.ds(g * 16, 16)] = idx_v[pl.ds(g * 16, 16)] + off
            return 0
        lax.fori_loop(0, B_PER_W // 16, off_body, 0)
        for t, tab in enumerate((n1_hbm, n2_hbm, n3_hbm)):
            pltpu.async_copy(tab.at[idxo_v], buf_v, sem).wait()
            pltpu.sync_copy(
                buf_v,
                out_hbm.at[pl.ds(base, B_PER_W), pl.ds((t + 1) * EMB, EMB)])


_fgather = functools.partial(
    pl.kernel,
    out_type=[
        jax.ShapeDtypeStruct((B_OUT, 4 * EMB), jnp.float32),
        jax.ShapeDtypeStruct((B_OUT, 4 * EMB), jnp.float32),
        jax.ShapeDtypeStruct((B_OUT, 4 * EMB), jnp.float32),
    ],
    mesh=plsc.VectorSubcoreMesh(core_axis_name="c", subcore_axis_name="s"),
    scratch_types=[
        pltpu.VMEM((B_PER_W,), jnp.int32),
        pltpu.VMEM((B_PER_W,), jnp.int32),
        pltpu.VMEM((B_PER_W, EMB), jnp.float32),
        pltpu.SemaphoreType.DMA,
    ],
    compiler_params=pltpu.CompilerParams(use_tc_tiling_on_sc=False),
)(_fgather_body)


def kernel(users, pos_items, neg_items, adj_row, adj_col, adj_val,
           user_emb, item_emb, W_gc_0, b_gc_0, W_bi_0, b_bi_0,
           W_gc_1, b_gc_1, W_bi_1, b_bi_1, W_gc_2, b_gc_2, W_bi_2, b_bi_2):
    n_user = user_emb.shape[0]
    ego0 = jnp.concatenate([user_emb, item_emb], axis=0)
    ego = jnp.pad(ego0, ((0, N_PAD - N_NODE), (0, 0)))

    Wg = [W_gc_0, W_gc_1, W_gc_2]
    bg = [b_gc_0, b_gc_1, b_gc_2]
    Wb = [W_bi_0, W_bi_1, W_bi_2]
    bb = [b_bi_0, b_bi_1, b_bi_2]

    colq_all = adj_col[None, :] * 4 + jnp.arange(4, dtype=jnp.int32)[:, None]

    norms = []
    for k in range(3):
        side = _spmm_layer(ego, colq_all, adj_row, adj_val)
        ego, nrm = _dense_layer(side, ego, Wg[k], bg[k], Wb[k], bb[k])
        norms.append(nrm)

    return tuple(_fgather(user_emb, item_emb, norms[0], norms[1], norms[2],
                          users, pos_items, neg_items))


# speedup vs baseline: 1.0547x; 1.0547x over previous
import functools
import jax
import jax.numpy as jnp
from jax import lax
from jax.experimental import pallas as pl
from jax.experimental.pallas import tpu as pltpu
from jax.experimental.pallas import tpu_sc as plsc

N_NODE = 50000
N_PAD = 50176
EMB = 64
HALF = 32
NNZ = 800000
BN = 1024
NSUB = 16
E_PER_S = NNZ // NSUB
CH = 400
N_CHUNK = E_PER_S // CH
ROWS_PER_S = N_PAD // NSUB
ZB = 392


def _dense_body(side_ref, ego_ref, wg_ref, bg_ref, wb_ref, bb_ref,
                ego_out_ref, norm_out_ref):
    side = side_ref[...]
    ego = ego_ref[...]
    sum_emb = jnp.dot(side, wg_ref[...],
                      preferred_element_type=jnp.float32) + bg_ref[...]
    bi = jnp.dot(ego * side, wb_ref[...],
                 preferred_element_type=jnp.float32) + bb_ref[...]
    x = sum_emb + bi
    x = jnp.where(x > 0, x, 0.2 * x)
    ego_out_ref[...] = x
    nrm = jnp.maximum(
        jnp.sqrt(jnp.sum(x * x, axis=1, keepdims=True)), 1e-12)
    norm_out_ref[...] = x / nrm


def _dense_layer(side, ego, wg, bg, wb, bb):
    n = side.shape[0]
    grid = (n // BN,)
    row_spec = pl.BlockSpec((BN, EMB), lambda i: (i, 0))
    w_spec = pl.BlockSpec((EMB, EMB), lambda i: (0, 0))
    b_spec = pl.BlockSpec((1, EMB), lambda i: (0, 0))
    return pl.pallas_call(
        _dense_body,
        grid=grid,
        in_specs=[row_spec, row_spec, w_spec, b_spec, w_spec, b_spec],
        out_specs=[row_spec, row_spec],
        out_shape=[
            jax.ShapeDtypeStruct((n, EMB), jnp.float32),
            jax.ShapeDtypeStruct((n, EMB), jnp.float32),
        ],
    )(side, ego, wg, bg, wb, bb)


NSLOT = 4


def _spmm_body(ego4_hbm, colq_hbm, row_hbm, val_hbm, out_hbm,
               col_v, dest_v, val_v, rows_v, zz_v, acc_sh,
               sem_t0, sem_t1, sem_t2, sem_t3,
               sem_g0, sem_g1, sem_g2, sem_g3,
               sem_s0, sem_s1, sem_s2, sem_s3):
    c = lax.axis_index("c")
    s = lax.axis_index("s")
    row0 = s * ROWS_PER_S
    ebase = s * E_PER_S
    sem_t = (sem_t0, sem_t1, sem_t2, sem_t3)
    sem_g = (sem_g0, sem_g1, sem_g2, sem_g3)
    sem_s = (sem_s0, sem_s1, sem_s2, sem_s3)

    for p in range(2):
        q = c * 2 + p

        def zero_body(i, _):
            zz_v[i, 0:16] = jnp.zeros((16,), jnp.float32)
            return 0
        lax.fori_loop(0, ZB, zero_body, 0)

        def zcopy_body(t, _):
            pltpu.sync_copy(zz_v, acc_sh.at[pl.ds(row0 + t * ZB, ZB)])
            return 0
        lax.fori_loop(0, ROWS_PER_S // ZB, zcopy_body, 0)
        plsc.subcore_barrier()

        def wait_scatter(sl):
            pltpu.make_async_copy(row_hbm.at[pl.ds(0, CH)],
                                  rows_v.at[sl], sem_s[sl]).wait()

        def stage(i, sl):
            off = ebase + i * CH
            pltpu.async_copy(colq_hbm.at[q, pl.ds(off, CH)], col_v.at[sl],
                             sem_t[sl])
            pltpu.async_copy(row_hbm.at[pl.ds(off, CH)], dest_v.at[sl],
                             sem_t[sl])
            pltpu.async_copy(val_hbm.at[pl.ds(off, CH)], val_v.at[sl],
                             sem_t[sl])

        def gather_start(sl):
            for dst in (col_v, dest_v, val_v):
                pltpu.make_async_copy(row_hbm.at[pl.ds(0, CH)],
                                      dst.at[sl], sem_t[sl]).wait()
            pltpu.async_copy(ego4_hbm.at[col_v.at[sl]], rows_v.at[sl],
                             sem_g[sl])

        def consume(sl):
            pltpu.make_async_copy(row_hbm.at[pl.ds(0, CH)],
                                  rows_v.at[sl], sem_g[sl]).wait()

            @plsc.parallel_loop(0, CH // 16, 1, unroll=2)
            def _(g):
                e0 = g * 16
                vals16 = val_v[sl, pl.ds(e0, 16)]
                for j in range(16):
                    rows_v[sl, e0 + j, 0:16] = (
                        rows_v[sl, e0 + j, 0:16] * vals16[j])

            pltpu.async_copy(rows_v.at[sl], acc_sh.at[dest_v.at[sl]],
                             sem_s[sl], add=True)

        stage(0, 0)
        stage(1, 1)
        gather_start(0)

        def ring_body(t, _):
            i0 = t * NSLOT
            for sl in range(NSLOT):
                i = i0 + sl
                nsl = (sl + 1) % NSLOT
                psl = (sl + 2) % NSLOT

                @pl.when(i + 2 < N_CHUNK)
                def _():
                    @pl.when(i >= 2)
                    def _():
                        wait_scatter(psl)
                    stage(i + 2, psl)

                @pl.when(i + 1 < N_CHUNK)
                def _():
                    gather_start(nsl)

                @pl.when(i < N_CHUNK)
                def _():
                    consume(sl)
            return 0

        lax.fori_loop(0, (N_CHUNK + NSLOT - 1) // NSLOT, ring_body, 0)
        for sl in range(NSLOT):
            wait_scatter(sl)
        plsc.subcore_barrier()

        pltpu.sync_copy(acc_sh.at[pl.ds(row0, ROWS_PER_S)],
                        out_hbm.at[pl.ds(row0, ROWS_PER_S), q])
        plsc.subcore_barrier()


_spmm = functools.partial(
    pl.kernel,
    out_type=jax.ShapeDtypeStruct((N_PAD, 4, 16), jnp.float32),
    mesh=plsc.VectorSubcoreMesh(core_axis_name="c", subcore_axis_name="s"),
    scratch_types=[
        pltpu.VMEM((NSLOT, CH), jnp.int32),
        pltpu.VMEM((NSLOT, CH), jnp.int32),
        pltpu.VMEM((NSLOT, CH), jnp.float32),
        pltpu.VMEM((NSLOT, CH, 16), jnp.float32),
        pltpu.VMEM((ZB, 16), jnp.float32),
        pltpu.VMEM_SHARED((N_PAD, 16), jnp.float32),
    ] + [pltpu.SemaphoreType.DMA] * 12,
    compiler_params=pltpu.CompilerParams(use_tc_tiling_on_sc=False),
)(_spmm_body)


def _spmm_layer(ego, colq_all, adj_row, adj_val):
    ego4 = ego.reshape(4 * N_PAD, 16)
    out = _spmm(ego4, colq_all, adj_row, adj_val)
    return out.reshape(N_PAD, EMB)


B_OUT = 4096
B_PER_W = B_OUT // 32


def _fgather_body(uemb_hbm, iemb_hbm, n1_hbm, n2_hbm, n3_hbm,
                  users_hbm, pos_hbm, neg_hbm,
                  ug_hbm, pg_hbm, ng_hbm,
                  idx_v, idxo_v, buf_v, sem):
    c = lax.axis_index("c")
    s = lax.axis_index("s")
    wid = s * 2 + c
    base = wid * B_PER_W

    for idx_hbm, out_hbm, is_item in ((users_hbm, ug_hbm, False),
                                      (pos_hbm, pg_hbm, True),
                                      (neg_hbm, ng_hbm, True)):
        pltpu.sync_copy(idx_hbm.at[pl.ds(base, B_PER_W)], idx_v)
        emb = iemb_hbm if is_item else uemb_hbm
        pltpu.async_copy(emb.at[idx_v], buf_v, sem).wait()
        pltpu.sync_copy(buf_v, out_hbm.at[pl.ds(base, B_PER_W), pl.ds(0, EMB)])
        off = 25000 if is_item else 0

        def off_body(g, _):
            idxo_v[pl.ds(g * 16, 16)] = idx_v[pl.ds(g * 16, 16)] + off
            return 0
        lax.fori_loop(0, B_PER_W // 16, off_body, 0)
        for t, tab in enumerate((n1_hbm, n2_hbm, n3_hbm)):
            pltpu.async_copy(tab.at[idxo_v], buf_v, sem).wait()
            pltpu.sync_copy(
                buf_v,
                out_hbm.at[pl.ds(base, B_PER_W), pl.ds((t + 1) * EMB, EMB)])


_fgather = functools.partial(
    pl.kernel,
    out_type=[
        jax.ShapeDtypeStruct((B_OUT, 4 * EMB), jnp.float32),
        jax.ShapeDtypeStruct((B_OUT, 4 * EMB), jnp.float32),
        jax.ShapeDtypeStruct((B_OUT, 4 * EMB), jnp.float32),
    ],
    mesh=plsc.VectorSubcoreMesh(core_axis_name="c", subcore_axis_name="s"),
    scratch_types=[
        pltpu.VMEM((B_PER_W,), jnp.int32),
        pltpu.VMEM((B_PER_W,), jnp.int32),
        pltpu.VMEM((B_PER_W, EMB), jnp.float32),
        pltpu.SemaphoreType.DMA,
    ],
    compiler_params=pltpu.CompilerParams(use_tc_tiling_on_sc=False),
)(_fgather_body)


def kernel(users, pos_items, neg_items, adj_row, adj_col, adj_val,
           user_emb, item_emb, W_gc_0, b_gc_0, W_bi_0, b_bi_0,
           W_gc_1, b_gc_1, W_bi_1, b_bi_1, W_gc_2, b_gc_2, W_bi_2, b_bi_2):
    n_user = user_emb.shape[0]
    ego0 = jnp.concatenate([user_emb, item_emb], axis=0)
    ego = jnp.pad(ego0, ((0, N_PAD - N_NODE), (0, 0)))

    Wg = [W_gc_0, W_gc_1, W_gc_2]
    bg = [b_gc_0, b_gc_1, b_gc_2]
    Wb = [W_bi_0, W_bi_1, W_bi_2]
    bb = [b_bi_0, b_bi_1, b_bi_2]

    colq_all = adj_col[None, :] * 4 + jnp.arange(4, dtype=jnp.int32)[:, None]

    norms = []
    for k in range(3):
        side = _spmm_layer(ego, colq_all, adj_row, adj_val)
        ego, nrm = _dense_layer(side, ego, Wg[k], bg[k], Wb[k], bb[k])
        norms.append(nrm)

    return tuple(_fgather(user_emb, item_emb, norms[0], norms[1], norms[2],
                          users, pos_items, neg_items))
